# speedup vs baseline: 1.0660x; 1.0660x over previous
import functools
import jax
import jax.numpy as jnp
from jax.experimental import pallas as pl
from jax.experimental.pallas import tpu as pltpu

N = 10000
E = 160000
G = 100
EPS = 1e-5


def _head_kernel(g_ref, w2_ref, b2_ref, wl_ref, bl_ref, out_ref):
    g = g_ref[...]
    h2 = jnp.maximum(jnp.dot(g, w2_ref[...], preferred_element_type=jnp.float32)
                     + b2_ref[...], 0.0)
    o = jnp.dot(h2, wl_ref[...], preferred_element_type=jnp.float32) + bl_ref[...]
    m = jnp.max(o[:, :1], axis=1, keepdims=True)
    e = jnp.exp(o - m)
    out_ref[...] = e / jnp.sum(e[:, :1], axis=1, keepdims=True)


def _head(g, p_seq2, p_lin):
    w2 = p_seq2["w"].T
    b2 = p_seq2["b"][None, :]
    wl = jnp.pad(p_lin["w"].T, ((0, 0), (0, 127)))
    bl = jnp.pad(p_lin["b"][None, :], ((0, 0), (0, 127)))
    out = pl.pallas_call(
        _head_kernel,
        out_shape=jax.ShapeDtypeStruct((G, 128), jnp.float32),
    )(g, w2, b2, wl, bl)
    return out[:, :1]


def _edge_conv(p, xin, src, dst, inv_cnt):
    W1 = p["lin1"]["w"]
    din = xin.shape[1]
    A = W1[:, :din]
    B = W1[:, din:]
    u = xin @ (A - B).T
    v = xin @ B.T
    y1 = u[dst] + v[src]
    s1 = y1.sum(0)
    q1 = (y1 * y1).sum(0)
    mu = s1 / E
    var = q1 / E - mu * mu
    sc = p["bn1"]["g"] * jax.lax.rsqrt(var + EPS)
    sh = p["bn1"]["b"] - mu * sc
    h = jnp.maximum(y1 * sc + sh, 0.0)
    y2 = h @ p["lin2"]["w"].T
    s2 = y2.sum(0)
    q2 = (y2 * y2).sum(0)
    mu2 = s2 / E
    var2 = q2 / E - mu2 * mu2
    sc2 = p["bn2"]["g"] * jax.lax.rsqrt(var2 + EPS)
    sh2 = p["bn2"]["b"] - mu2 * sc2
    h2 = jnp.maximum(y2 * sc2 + sh2, 0.0)
    ssum = jax.ops.segment_sum(h2, dst, num_segments=N)
    return ssum * inv_cnt[:, None]


def _bnN(p, z):
    mu = z.mean(0)
    var = z.var(0)
    sc = p["g"] * jax.lax.rsqrt(var + EPS)
    sh = p["b"] - mu * sc
    return z * sc + sh


def kernel(x, params, edge_index, batch):
    src = edge_index[0]
    dst = edge_index[1]
    cnt = jax.ops.segment_sum(jnp.ones((E,), jnp.float32), dst, num_segments=N)
    inv_cnt = 1.0 / jnp.maximum(cnt, 1.0)

    x1 = jnp.maximum(_bnN(params["sc1"]["bn"], x @ params["sc1"]["lin"]["w"].T)
                     + _edge_conv(params["conv1"], x, src, dst, inv_cnt), 0.0)
    x2 = jnp.maximum(_edge_conv(params["conv2"], x1, src, dst, inv_cnt) + x1, 0.0)
    x3 = jnp.maximum(_bnN(params["sc3"]["bn"], x2 @ params["sc3"]["lin"]["w"].T)
                     + _edge_conv(params["conv3"], x2, src, dst, inv_cnt), 0.0)
    x4 = jnp.maximum(_edge_conv(params["conv4"], x3, src, dst, inv_cnt) + x3, 0.0)
    x5 = jnp.maximum(_bnN(params["sc5"]["bn"], x4 @ params["sc5"]["lin"]["w"].T)
                     + _edge_conv(params["conv5"], x4, src, dst, inv_cnt), 0.0)
    x6 = jnp.maximum(_edge_conv(params["conv6"], x5, src, dst, inv_cnt) + x5, 0.0)

    xc = jnp.concatenate([x1, x2, x3, x4, x5, x6], axis=1)
    z = xc @ params["seq1"]["lin"]["w"].T
    h = jnp.maximum(_bnN(params["seq1"]["bn"], z), 0.0)
    s = jax.ops.segment_sum(h, batch, num_segments=G)
    bcnt = jax.ops.segment_sum(jnp.ones((N,), jnp.float32), batch, num_segments=G)
    g = s * (1.0 / jnp.maximum(bcnt, 1.0))[:, None]
    return _head(g, params["seq2"]["lin"], params["lin"])


# speedup vs baseline: 2.2890x; 2.1473x over previous
import jax
import jax.numpy as jnp
from jax import lax
from jax.experimental import pallas as pl
from jax.experimental.pallas import tpu as pltpu
from jax.experimental.pallas import tpu_sc as plsc

N = 10000
E = 160000
G = 100
EPS = 1e-5
NC = 2
NS = 16
NW = NC * NS
EPW = E // NW
F32 = jnp.float32


def _mesh():
    return plsc.VectorSubcoreMesh(core_axis_name="c", subcore_axis_name="s",
                                  num_cores=NC, num_subcores=NS)


def _sc_gather(u, v, src, dst):
    ch = 200
    steps = EPW // ch

    def body(u_h, v_h, src_h, dst_h, gu_h, gv_h, idxd, idxs, bufu, bufv,
             semu, semv):
        wid = lax.axis_index("s") * NC + lax.axis_index("c")
        base = wid * EPW

        def step(i, carry):
            off = base + i * ch
            pltpu.sync_copy(dst_h.at[pl.ds(off, ch)], idxd)
            pltpu.sync_copy(src_h.at[pl.ds(off, ch)], idxs)
            cu = pltpu.async_copy(u_h.at[idxd], bufu, semu)
            cv = pltpu.async_copy(v_h.at[idxs], bufv, semv)
            cu.wait()
            cv.wait()
            pltpu.sync_copy(bufu, gu_h.at[pl.ds(off, ch)])
            pltpu.sync_copy(bufv, gv_h.at[pl.ds(off, ch)])
            return carry

        lax.fori_loop(0, steps, step, 0)

    return pl.kernel(
        body,
        out_type=[jax.ShapeDtypeStruct((E, 128), F32),
                  jax.ShapeDtypeStruct((E, 128), F32)],
        mesh=_mesh(),
        scratch_types=[pltpu.VMEM((ch,), jnp.int32),
                       pltpu.VMEM((ch,), jnp.int32),
                       pltpu.VMEM((ch, 128), F32),
                       pltpu.VMEM((ch, 128), F32),
                       pltpu.SemaphoreType.DMA,
                       pltpu.SemaphoreType.DMA],
    )(u, v, src, dst)


EP = 163840
EPW2 = EP // NW
NP = 10112
RPTN = NP // NS


def _sc_scatter(h2p, dstp, zrows):
    d = 128

    def body(h2_h, dst_h, z_h, out_h, idx, buf, sem, acc):
        c = lax.axis_index("c")
        s = lax.axis_index("s")
        base = (s * NC + c) * EPW2
        pltpu.sync_copy(z_h, acc.at[pl.ds(s * RPTN, RPTN)])
        plsc.subcore_barrier()

        def step(i, carry):
            off = base + i * 128
            pltpu.sync_copy(dst_h.at[pl.ds(off, 128)], idx)
            pltpu.sync_copy(h2_h.at[pl.ds(off, 128)], buf)
            pltpu.async_copy(buf, acc.at[idx], sem, add=True).wait()
            return carry

        lax.fori_loop(0, 40, step, 0)
        plsc.subcore_barrier()
        pltpu.sync_copy(acc.at[pl.ds(s * RPTN, RPTN)],
                        out_h.at[pl.ds(c * NP + s * RPTN, RPTN)])

    return pl.kernel(
        body,
        out_type=jax.ShapeDtypeStruct((2 * NP, d), F32),
        mesh=_mesh(),
        scratch_types=[pltpu.VMEM((128,), jnp.int32),
                       pltpu.VMEM((128, d), F32),
                       pltpu.SemaphoreType.DMA,
                       pltpu.VMEM_SHARED((NP, d), F32)],
    )(h2p, dstp, zrows)


def _sc_counts(dstp, ones_h_in, zrows):
    def body(dst_h, on_h, z_h, out_h, idx, onesbuf, sem, acc):
        c = lax.axis_index("c")
        s = lax.axis_index("s")
        base = (s * NC + c) * EPW2
        pltpu.sync_copy(on_h, onesbuf)
        pltpu.sync_copy(z_h, acc.at[pl.ds(s * RPTN, RPTN)])
        plsc.subcore_barrier()

        def step(i, carry):
            off = base + i * 128
            pltpu.sync_copy(dst_h.at[pl.ds(off, 128)], idx)
            pltpu.async_copy(onesbuf, acc.at[idx], sem, add=True).wait()
            return carry

        lax.fori_loop(0, 40, step, 0)
        plsc.subcore_barrier()
        pltpu.sync_copy(acc.at[pl.ds(s * RPTN, RPTN)],
                        out_h.at[pl.ds(c * NP + s * RPTN, RPTN)])

    return pl.kernel(
        body,
        out_type=jax.ShapeDtypeStruct((2 * NP, 128), F32),
        mesh=_mesh(),
        scratch_types=[pltpu.VMEM((128,), jnp.int32),
                       pltpu.VMEM((128, 128), F32),
                       pltpu.SemaphoreType.DMA,
                       pltpu.VMEM_SHARED((NP, 128), F32)],
    )(dstp, ones_h_in, zrows)


GP = 104


def _sc_pool(h0, h1, h2, batchp, zc, onesp):
    nchunk = NP // 128

    def body(h0_h, h1_h, h2_h, b_h, zc_h, on_h,
             o0_h, o1_h, o2_h, oc_h, idx, b0, b1, b2, ones_v,
             s0, s1, s2, sc, a0, a1, a2, ac):
        c = lax.axis_index("c")
        s = lax.axis_index("s")
        w = s * NC + c

        @pl.when(s == 0)
        def _():
            pltpu.sync_copy(zc_h, a0)
            pltpu.sync_copy(zc_h, a1)
            pltpu.sync_copy(zc_h, a2)
            pltpu.sync_copy(zc_h, ac)

        pltpu.sync_copy(on_h, ones_v)
        plsc.subcore_barrier()
        trip = 2 + (w < nchunk - 2 * NW).astype(jnp.int32)

        def step(j, carry):
            off = (w + NW * j) * 128
            pltpu.sync_copy(b_h.at[pl.ds(off, 128)], idx)
            pltpu.sync_copy(h0_h.at[pl.ds(off, 128)], b0)
            pltpu.sync_copy(h1_h.at[pl.ds(off, 128)], b1)
            pltpu.sync_copy(h2_h.at[pl.ds(off, 128)], b2)
            c0 = pltpu.async_copy(b0, a0.at[idx], s0, add=True)
            c1 = pltpu.async_copy(b1, a1.at[idx], s1, add=True)
            c2 = pltpu.async_copy(b2, a2.at[idx], s2, add=True)
            c3 = pltpu.async_copy(ones_v, ac.at[idx], sc, add=True)
            c0.wait()
            c1.wait()
            c2.wait()
            c3.wait()
            return carry

        lax.fori_loop(0, trip, step, 0)
        plsc.subcore_barrier()

        @pl.when(s == 0)
        def _():
            pltpu.sync_copy(a0, o0_h.at[pl.ds(c * GP, GP)])
            pltpu.sync_copy(a1, o1_h.at[pl.ds(c * GP, GP)])
            pltpu.sync_copy(a2, o2_h.at[pl.ds(c * GP, GP)])
            pltpu.sync_copy(ac, oc_h.at[pl.ds(c * GP, GP)])

    return pl.kernel(
        body,
        out_type=[jax.ShapeDtypeStruct((2 * GP, 128), F32)] * 4,
        mesh=_mesh(),
        scratch_types=[pltpu.VMEM((128,), jnp.int32)]
        + [pltpu.VMEM((128, 128), F32)] * 4
        + [pltpu.SemaphoreType.DMA] * 4
        + [pltpu.VMEM_SHARED((GP, 128), F32)] * 4,
    )(h0, h1, h2, batchp, zc, onesp)


def _stats_update(aff_r, st, i, steps, g_r, b_r, denom):
    @pl.when(i == 0)
    def _():
        aff_r[...] = st

    @pl.when(i > 0)
    def _():
        aff_r[...] = aff_r[...] + st

    @pl.when(i == steps - 1)
    def _():
        a = aff_r[...]
        mu = a[0:1] / denom
        var = a[1:2] / denom - mu * mu
        sc = g_r[...] * lax.rsqrt(var + EPS)
        sh = b_r[...] - mu * sc
        aff_r[...] = jnp.concatenate([sc, sh], 0)


def _tc_addstats(gu, gv, g, b, d):
    be = 4000
    steps = E // be

    def body(gu_r, gv_r, g_r, b_r, y_r, aff_r):
        i = pl.program_id(0)
        y = gu_r[...][:, :d] + gv_r[...][:, :d]
        y_r[...] = y
        st = jnp.concatenate([jnp.sum(y, 0, keepdims=True),
                              jnp.sum(y * y, 0, keepdims=True)], 0)
        _stats_update(aff_r, st, i, steps, g_r, b_r, float(E))

    return pl.pallas_call(
        body,
        grid=(steps,),
        in_specs=[pl.BlockSpec((be, 128), lambda i: (i, 0)),
                  pl.BlockSpec((be, 128), lambda i: (i, 0)),
                  pl.BlockSpec((1, d), lambda i: (0, 0)),
                  pl.BlockSpec((1, d), lambda i: (0, 0))],
        out_specs=[pl.BlockSpec((be, d), lambda i: (i, 0)),
                   pl.BlockSpec((2, d), lambda i: (0, 0))],
        out_shape=[jax.ShapeDtypeStruct((E, d), F32),
                   jax.ShapeDtypeStruct((2, d), F32)],
    )(gu, gv, g.reshape(1, d), b.reshape(1, d))


def _tc_mm2(y1, aff1, w2t, g, b):
    d = y1.shape[1]
    dn = w2t.shape[1]
    be = 4000
    steps = E // be

    def body(y1_r, a1_r, w_r, g_r, b_r, y2_r, aff_r):
        i = pl.program_id(0)
        a = a1_r[...]
        h = jnp.maximum(y1_r[...] * a[0:1] + a[1:2], 0.0)
        y = jnp.dot(h, w_r[...], preferred_element_type=F32)
        y2_r[...] = y
        st = jnp.concatenate([jnp.sum(y, 0, keepdims=True),
                              jnp.sum(y * y, 0, keepdims=True)], 0)
        _stats_update(aff_r, st, i, steps, g_r, b_r, float(E))

    return pl.pallas_call(
        body,
        grid=(steps,),
        in_specs=[pl.BlockSpec((be, d), lambda i: (i, 0)),
                  pl.BlockSpec((2, d), lambda i: (0, 0)),
                  pl.BlockSpec((d, dn), lambda i: (0, 0)),
                  pl.BlockSpec((1, dn), lambda i: (0, 0)),
                  pl.BlockSpec((1, dn), lambda i: (0, 0))],
        out_specs=[pl.BlockSpec((be, dn), lambda i: (i, 0)),
                   pl.BlockSpec((2, dn), lambda i: (0, 0))],
        out_shape=[jax.ShapeDtypeStruct((E, dn), F32),
                   jax.ShapeDtypeStruct((2, dn), F32)],
    )(y1, aff1, w2t, g.reshape(1, dn), b.reshape(1, dn))


def _tc_act(y, aff, pad_to=None):
    m, d = y.shape
    do = pad_to or d
    bm = 4000 if m % 4000 == 0 else 2000
    steps = m // bm

    def body(y_r, a_r, o_r):
        a = a_r[...]
        h = jnp.maximum(y_r[...] * a[0:1] + a[1:2], 0.0)
        if do > d:
            h = jnp.concatenate([h, jnp.zeros((bm, do - d), F32)], axis=1)
        o_r[...] = h

    return pl.pallas_call(
        body,
        grid=(steps,),
        in_specs=[pl.BlockSpec((bm, d), lambda i: (i, 0)),
                  pl.BlockSpec((2, d), lambda i: (0, 0))],
        out_specs=pl.BlockSpec((bm, do), lambda i: (i, 0)),
        out_shape=jax.ShapeDtypeStruct((m, do), F32),
    )(y, aff)


def _agg(p_r, c_r, d):
    pr = p_r[...]
    agg = pr[0, :, :d] + pr[1, :, :d]
    cc = c_r[...]
    cnt = cc[0, :, 0:1] + cc[1, :, 0:1]
    return agg * (1.0 / jnp.maximum(cnt, 1.0))


def _tc_k1(xp, a1t, b1t, ws1t, g1, b1):
    dn = ws1t.shape[1]
    bn = 2000
    steps = N // bn

    def body(x_r, at_r, bt_r, wst_r, g_r, b_r, u_r, v_r, s_r, aff_r):
        i = pl.program_id(0)
        x = x_r[...]
        u_r[...] = jnp.dot(x, at_r[...], preferred_element_type=F32)
        v_r[...] = jnp.dot(x, bt_r[...], preferred_element_type=F32)
        s = jnp.dot(x, wst_r[...], preferred_element_type=F32)
        s_r[...] = s
        st = jnp.concatenate([jnp.sum(s, 0, keepdims=True),
                              jnp.sum(s * s, 0, keepdims=True)], 0)
        _stats_update(aff_r, st, i, steps, g_r, b_r, float(N))

    return pl.pallas_call(
        body,
        grid=(steps,),
        in_specs=[pl.BlockSpec((bn, 8), lambda i: (i, 0)),
                  pl.BlockSpec((8, 128), lambda i: (0, 0)),
                  pl.BlockSpec((8, 128), lambda i: (0, 0)),
                  pl.BlockSpec((8, dn), lambda i: (0, 0)),
                  pl.BlockSpec((1, dn), lambda i: (0, 0)),
                  pl.BlockSpec((1, dn), lambda i: (0, 0))],
        out_specs=[pl.BlockSpec((bn, 128), lambda i: (i, 0)),
                   pl.BlockSpec((bn, 128), lambda i: (i, 0)),
                   pl.BlockSpec((bn, dn), lambda i: (i, 0)),
                   pl.BlockSpec((2, dn), lambda i: (0, 0))],
        out_shape=[jax.ShapeDtypeStruct((N, 128), F32),
                   jax.ShapeDtypeStruct((N, 128), F32),
                   jax.ShapeDtypeStruct((N, dn), F32),
                   jax.ShapeDtypeStruct((2, dn), F32)],
    )(xp, a1t, b1t, ws1t, g1.reshape(1, dn), b1.reshape(1, dn))


def _tc_j_short(p, cnts, s_arr, aff_s, at, bt):
    d = s_arr.shape[1]
    bn = 2000
    steps = N // bn

    def body(p_r, c_r, s_r, a_r, at_r, bt_r, x_r, u_r, v_r):
        a = a_r[...]
        xk = jnp.maximum(s_r[...] * a[0:1] + a[1:2] + _agg(p_r, c_r, d), 0.0)
        x_r[...] = xk
        u_r[...] = jnp.dot(xk, at_r[...], preferred_element_type=F32)
        v_r[...] = jnp.dot(xk, bt_r[...], preferred_element_type=F32)

    return pl.pallas_call(
        body,
        grid=(steps,),
        in_specs=[pl.BlockSpec((2, bn, 128), lambda i: (0, i, 0)),
                  pl.BlockSpec((2, bn, 128), lambda i: (0, i, 0)),
                  pl.BlockSpec((bn, d), lambda i: (i, 0)),
                  pl.BlockSpec((2, d), lambda i: (0, 0)),
                  pl.BlockSpec((d, 128), lambda i: (0, 0)),
                  pl.BlockSpec((d, 128), lambda i: (0, 0))],
        out_specs=[pl.BlockSpec((bn, d), lambda i: (i, 0)),
                   pl.BlockSpec((bn, 128), lambda i: (i, 0)),
                   pl.BlockSpec((bn, 128), lambda i: (i, 0))],
        out_shape=[jax.ShapeDtypeStruct((N, d), F32),
                   jax.ShapeDtypeStruct((N, 128), F32),
                   jax.ShapeDtypeStruct((N, 128), F32)],
    )(p.reshape(2, NP, 128), cnts.reshape(2, NP, 128), s_arr, aff_s, at, bt)


def _tc_j_res(p, cnts, xprev, at, bt, wst, gs, bs):
    d = xprev.shape[1]
    dn = wst.shape[1]
    bn = 2000
    steps = N // bn

    def body(p_r, c_r, xp_r, at_r, bt_r, wst_r, g_r, b_r,
             x_r, u_r, v_r, s_r, aff_r):
        i = pl.program_id(0)
        xk = jnp.maximum(_agg(p_r, c_r, d) + xp_r[...], 0.0)
        x_r[...] = xk
        u_r[...] = jnp.dot(xk, at_r[...], preferred_element_type=F32)
        v_r[...] = jnp.dot(xk, bt_r[...], preferred_element_type=F32)
        s = jnp.dot(xk, wst_r[...], preferred_element_type=F32)
        s_r[...] = s
        st = jnp.concatenate([jnp.sum(s, 0, keepdims=True),
                              jnp.sum(s * s, 0, keepdims=True)], 0)
        _stats_update(aff_r, st, i, steps, g_r, b_r, float(N))

    return pl.pallas_call(
        body,
        grid=(steps,),
        in_specs=[pl.BlockSpec((2, bn, 128), lambda i: (0, i, 0)),
                  pl.BlockSpec((2, bn, 128), lambda i: (0, i, 0)),
                  pl.BlockSpec((bn, d), lambda i: (i, 0)),
                  pl.BlockSpec((d, 128), lambda i: (0, 0)),
                  pl.BlockSpec((d, 128), lambda i: (0, 0)),
                  pl.BlockSpec((d, dn), lambda i: (0, 0)),
                  pl.BlockSpec((1, dn), lambda i: (0, 0)),
                  pl.BlockSpec((1, dn), lambda i: (0, 0))],
        out_specs=[pl.BlockSpec((bn, d), lambda i: (i, 0)),
                   pl.BlockSpec((bn, 128), lambda i: (i, 0)),
                   pl.BlockSpec((bn, 128), lambda i: (i, 0)),
                   pl.BlockSpec((bn, dn), lambda i: (i, 0)),
                   pl.BlockSpec((2, dn), lambda i: (0, 0))],
        out_shape=[jax.ShapeDtypeStruct((N, d), F32),
                   jax.ShapeDtypeStruct((N, 128), F32),
                   jax.ShapeDtypeStruct((N, 128), F32),
                   jax.ShapeDtypeStruct((N, dn), F32),
                   jax.ShapeDtypeStruct((2, dn), F32)],
    )(p.reshape(2, NP, 128), cnts.reshape(2, NP, 128), xprev, at, bt, wst,
      gs.reshape(1, dn), bs.reshape(1, dn))


def _tc_j6(p, cnts, xs, wzs, gz, bz):
    d = xs[4].shape[1]
    dz = wzs[0].shape[1]
    bn = 2000
    steps = N // bn

    def body(p_r, c_r, x1_r, x2_r, x3_r, x4_r, x5_r,
             w1_r, w2_r, w3_r, w4_r, w5_r, w6_r, g_r, b_r, z_r, aff_r):
        i = pl.program_id(0)
        x6 = jnp.maximum(_agg(p_r, c_r, d) + x5_r[...], 0.0)
        z = (jnp.dot(x1_r[...], w1_r[...], preferred_element_type=F32)
             + jnp.dot(x2_r[...], w2_r[...], preferred_element_type=F32)
             + jnp.dot(x3_r[...], w3_r[...], preferred_element_type=F32)
             + jnp.dot(x4_r[...], w4_r[...], preferred_element_type=F32)
             + jnp.dot(x5_r[...], w5_r[...], preferred_element_type=F32)
             + jnp.dot(x6, w6_r[...], preferred_element_type=F32))
        z_r[...] = z
        st = jnp.concatenate([jnp.sum(z, 0, keepdims=True),
                              jnp.sum(z * z, 0, keepdims=True)], 0)
        _stats_update(aff_r, st, i, steps, g_r, b_r, float(N))

    x1, x2, x3, x4, x5 = xs
    ds = [x.shape[1] for x in xs]
    return pl.pallas_call(
        body,
        grid=(steps,),
        in_specs=[pl.BlockSpec((2, bn, 128), lambda i: (0, i, 0)),
                  pl.BlockSpec((2, bn, 128), lambda i: (0, i, 0))]
        + [pl.BlockSpec((bn, dd), lambda i: (i, 0)) for dd in ds]
        + [pl.BlockSpec((dd, dz), lambda i: (0, 0)) for dd in ds + [d]]
        + [pl.BlockSpec((1, dz), lambda i: (0, 0)),
           pl.BlockSpec((1, dz), lambda i: (0, 0))],
        out_specs=[pl.BlockSpec((bn, dz), lambda i: (i, 0)),
                   pl.BlockSpec((2, dz), lambda i: (0, 0))],
        out_shape=[jax.ShapeDtypeStruct((N, dz), F32),
                   jax.ShapeDtypeStruct((2, dz), F32)],
    )(p.reshape(2, NP, 128), cnts.reshape(2, NP, 128), x1, x2, x3, x4, x5,
      *wzs, gz.reshape(1, dz), bz.reshape(1, dz))


def _tc_head(pps, pc, w2t, b2, wlt, bl):
    def body(p0_r, p1_r, p2_r, pc_r, w2_r, b2_r, wl_r, bl_r, out_r):
        pr = jnp.concatenate([p0_r[...], p1_r[...], p2_r[...]], axis=2)
        cc = pc_r[...]
        cnt = cc[0, :, 0:1] + cc[1, :, 0:1]
        g = (pr[0] + pr[1]) * (1.0 / jnp.maximum(cnt, 1.0))
        h2 = jnp.maximum(jnp.dot(g, w2_r[...], preferred_element_type=F32)
                         + b2_r[...], 0.0)
        o = jnp.dot(h2, wl_r[...], preferred_element_type=F32) + bl_r[...]
        m = jnp.max(o[:, :1], axis=1, keepdims=True)
        e = jnp.exp(o - m)
        out_r[...] = e / jnp.sum(e[:, :1], axis=1, keepdims=True)

    return pl.pallas_call(
        body,
        out_shape=jax.ShapeDtypeStruct((GP, 128), F32),
    )(pps[0].reshape(2, GP, 128), pps[1].reshape(2, GP, 128),
      pps[2].reshape(2, GP, 128), pc.reshape(2, GP, 128), w2t, b2, wlt, bl)


def _edge_layer(u, v, src, dst, dstp, p, zrows):
    d = p["lin2"]["w"].shape[0]
    gu, gv = _sc_gather(u, v, src, dst)
    y1, aff1 = _tc_addstats(gu, gv, p["bn1"]["g"], p["bn1"]["b"], d)
    y2, aff2 = _tc_mm2(y1, aff1, p["lin2"]["w"].T, p["bn2"]["g"], p["bn2"]["b"])
    h2 = _tc_act(y2, aff2, pad_to=128)
    h2p = jnp.pad(h2, ((0, EP - E), (0, 0)))
    return _sc_scatter(h2p, dstp, zrows)


def _split_w1(p, din):
    w1 = p["lin1"]["w"]
    dout = w1.shape[0]
    a = w1[:, :din]
    b = w1[:, din:]
    pad = ((0, 0), (0, 128 - dout))
    return jnp.pad((a - b).T, pad), jnp.pad(b.T, pad)


def kernel(x, params, edge_index, batch):
    src = edge_index[0]
    dst = edge_index[1]

    z128 = jnp.zeros((RPTN, 128), F32)
    dstp = jnp.pad(dst, (0, EP - E), constant_values=N)

    cnts = _sc_counts(dstp, jnp.ones((128, 128), F32), z128)

    xp = jnp.pad(x, ((0, 0), (0, 5)))
    a1t, b1t = _split_w1(params["conv1"], 3)
    a1t = jnp.pad(a1t, ((0, 5), (0, 0)))
    b1t = jnp.pad(b1t, ((0, 5), (0, 0)))
    ws1t = jnp.pad(params["sc1"]["lin"]["w"].T, ((0, 5), (0, 0)))
    u1, v1, s1, aff_s1 = _tc_k1(xp, a1t, b1t, ws1t,
                                params["sc1"]["bn"]["g"],
                                params["sc1"]["bn"]["b"])
    p1 = _edge_layer(u1, v1, src, dst, dstp, params["conv1"], z128)

    a2t, b2t = _split_w1(params["conv2"], 32)
    x1, u2, v2 = _tc_j_short(p1, cnts, s1, aff_s1, a2t, b2t)
    p2 = _edge_layer(u2, v2, src, dst, dstp, params["conv2"], z128)

    a3t, b3t = _split_w1(params["conv3"], 32)
    x2, u3, v3, s3, aff_s3 = _tc_j_res(p2, cnts, x1, a3t, b3t,
                                       params["sc3"]["lin"]["w"].T,
                                       params["sc3"]["bn"]["g"],
                                       params["sc3"]["bn"]["b"])
    p3 = _edge_layer(u3, v3, src, dst, dstp, params["conv3"], z128)

    a4t, b4t = _split_w1(params["conv4"], 64)
    x3, u4, v4 = _tc_j_short(p3, cnts, s3, aff_s3, a4t, b4t)
    p4 = _edge_layer(u4, v4, src, dst, dstp, params["conv4"], z128)

    a5t, b5t = _split_w1(params["conv5"], 64)
    x4, u5, v5, s5, aff_s5 = _tc_j_res(p4, cnts, x3, a5t, b5t,
                                       params["sc5"]["lin"]["w"].T,
                                       params["sc5"]["bn"]["g"],
                                       params["sc5"]["bn"]["b"])
    p5 = _edge_layer(u5, v5, src, dst, dstp, params["conv5"], z128)

    a6t, b6t = _split_w1(params["conv6"], 128)
    x5, u6, v6 = _tc_j_short(p5, cnts, s5, aff_s5, a6t, b6t)
    p6 = _edge_layer(u6, v6, src, dst, dstp, params["conv6"], z128)

    wseq = params["seq1"]["lin"]["w"]
    offs = [0, 32, 64, 128, 192, 320, 448]
    wzs = [wseq[:, offs[k]:offs[k + 1]].T for k in range(6)]
    z, aff_z = _tc_j6(p6, cnts, [x1, x2, x3, x4, x5], wzs,
                      params["seq1"]["bn"]["g"], params["seq1"]["bn"]["b"])
    h = _tc_act(z, aff_z)

    hp = jnp.pad(h, ((0, NP - N), (0, 0)))
    batchp = jnp.pad(batch, (0, NP - N), constant_values=G)
    zc = jnp.zeros((GP, 128), F32)
    onesp = jnp.ones((128, 128), F32)
    pp0, pp1, pp2, pc = _sc_pool(hp[:, :128], hp[:, 128:256], hp[:, 256:],
                                 batchp, zc, onesp)

    w2t = params["seq2"]["lin"]["w"].T
    b2 = params["seq2"]["lin"]["b"].reshape(1, 256)
    wlt = jnp.pad(params["lin"]["w"].T, ((0, 0), (0, 127)))
    bl = jnp.pad(params["lin"]["b"].reshape(1, 1), ((0, 0), (0, 127)))
    out = _tc_head((pp0, pp1, pp2), pc, w2t, b2, wlt, bl)
    return out[:G, :1]


# speedup vs baseline: 2.6290x; 1.1485x over previous
import jax
import jax.numpy as jnp
from jax import lax
from jax.experimental import pallas as pl
from jax.experimental.pallas import tpu as pltpu
from jax.experimental.pallas import tpu_sc as plsc

N = 10000
E = 160000
G = 100
EPS = 1e-5
NC = 2
NS = 16
NW = NC * NS
EPW = E // NW
F32 = jnp.float32


def _mesh():
    return plsc.VectorSubcoreMesh(core_axis_name="c", subcore_axis_name="s",
                                  num_cores=NC, num_subcores=NS)


def _sc_gather(u, v, src, dst):
    ch = 200
    npair = EPW // (2 * ch)

    def body(u_h, v_h, src_h, dst_h, gu_h, gv_h,
             ida, isa, bua, bva, idb, isb, bub, bvb,
             sga, sgb, sgc, sgd, swa, swb, swc, swd):
        wid = lax.axis_index("s") * NC + lax.axis_index("c")
        base = wid * EPW

        def chunk_fetch(off, idxd, idxs, bufu, bufv, s1, s2):
            pltpu.sync_copy(dst_h.at[pl.ds(off, ch)], idxd)
            pltpu.sync_copy(src_h.at[pl.ds(off, ch)], idxs)
            cu = pltpu.async_copy(u_h.at[idxd], bufu, s1)
            cv = pltpu.async_copy(v_h.at[idxs], bufv, s2)
            return cu, cv

        def step(i, carry):
            offa = base + (2 * i) * ch
            offb = base + (2 * i + 1) * ch

            @pl.when(i > 0)
            def _():
                pltpu.make_async_copy(bua, gu_h.at[pl.ds(offa, ch)], swa).wait()
                pltpu.make_async_copy(bva, gv_h.at[pl.ds(offa, ch)], swb).wait()

            cua, cva = chunk_fetch(offa, ida, isa, bua, bva, sga, sgb)

            @pl.when(i > 0)
            def _():
                pltpu.make_async_copy(bub, gu_h.at[pl.ds(offb, ch)], swc).wait()
                pltpu.make_async_copy(bvb, gv_h.at[pl.ds(offb, ch)], swd).wait()

            cub, cvb = chunk_fetch(offb, idb, isb, bub, bvb, sgc, sgd)
            cua.wait()
            cva.wait()
            pltpu.async_copy(bua, gu_h.at[pl.ds(offa, ch)], swa)
            pltpu.async_copy(bva, gv_h.at[pl.ds(offa, ch)], swb)
            cub.wait()
            cvb.wait()
            pltpu.async_copy(bub, gu_h.at[pl.ds(offb, ch)], swc)
            pltpu.async_copy(bvb, gv_h.at[pl.ds(offb, ch)], swd)
            return carry

        lax.fori_loop(0, npair, step, 0)
        offz = base
        pltpu.make_async_copy(bua, gu_h.at[pl.ds(offz, ch)], swa).wait()
        pltpu.make_async_copy(bva, gv_h.at[pl.ds(offz, ch)], swb).wait()
        pltpu.make_async_copy(bub, gu_h.at[pl.ds(offz, ch)], swc).wait()
        pltpu.make_async_copy(bvb, gv_h.at[pl.ds(offz, ch)], swd).wait()
        offt = base + 24 * ch
        cu, cv = chunk_fetch(offt, ida, isa, bua, bva, sga, sgb)
        cu.wait()
        cv.wait()
        pltpu.sync_copy(bua, gu_h.at[pl.ds(offt, ch)])
        pltpu.sync_copy(bva, gv_h.at[pl.ds(offt, ch)])

    return pl.kernel(
        body,
        out_type=[jax.ShapeDtypeStruct((E, 128), F32),
                  jax.ShapeDtypeStruct((E, 128), F32)],
        mesh=_mesh(),
        scratch_types=[pltpu.VMEM((ch,), jnp.int32),
                       pltpu.VMEM((ch,), jnp.int32),
                       pltpu.VMEM((ch, 128), F32),
                       pltpu.VMEM((ch, 128), F32),
                       pltpu.VMEM((ch,), jnp.int32),
                       pltpu.VMEM((ch,), jnp.int32),
                       pltpu.VMEM((ch, 128), F32),
                       pltpu.VMEM((ch, 128), F32)]
        + [pltpu.SemaphoreType.DMA] * 8,
    )(u, v, src, dst)


EP = 163840
EPW2 = EP // NW
NP = 10112
RPTN = NP // NS


def _sc_scatter(h2p, dstp, zrows):
    d = 128
    npair = 20

    def body(h2_h, dst_h, z_h, out_h, ixa, bfa, ixb, bfb, sfa, sfb, saa, sab,
             acc):
        c = lax.axis_index("c")
        s = lax.axis_index("s")
        base = (s * NC + c) * EPW2
        pltpu.sync_copy(z_h, acc.at[pl.ds(s * RPTN, RPTN)])
        plsc.subcore_barrier()

        def step(i, carry):
            offa = base + (2 * i) * 128
            offb = base + (2 * i + 1) * 128

            @pl.when(i > 0)
            def _():
                pltpu.make_async_copy(bfa, acc.at[ixa], saa).wait()

            pltpu.sync_copy(dst_h.at[pl.ds(offa, 128)], ixa)
            ca = pltpu.async_copy(h2_h.at[pl.ds(offa, 128)], bfa, sfa)

            @pl.when(i > 0)
            def _():
                pltpu.make_async_copy(bfb, acc.at[ixb], sab).wait()

            pltpu.sync_copy(dst_h.at[pl.ds(offb, 128)], ixb)
            cb = pltpu.async_copy(h2_h.at[pl.ds(offb, 128)], bfb, sfb)
            ca.wait()
            pltpu.async_copy(bfa, acc.at[ixa], saa, add=True)
            cb.wait()
            pltpu.async_copy(bfb, acc.at[ixb], sab, add=True)
            return carry

        lax.fori_loop(0, npair, step, 0)
        pltpu.make_async_copy(bfa, acc.at[ixa], saa).wait()
        pltpu.make_async_copy(bfb, acc.at[ixb], sab).wait()
        plsc.subcore_barrier()
        pltpu.sync_copy(acc.at[pl.ds(s * RPTN, RPTN)],
                        out_h.at[pl.ds(c * NP + s * RPTN, RPTN)])

    return pl.kernel(
        body,
        out_type=jax.ShapeDtypeStruct((2 * NP, d), F32),
        mesh=_mesh(),
        scratch_types=[pltpu.VMEM((128,), jnp.int32),
                       pltpu.VMEM((128, d), F32),
                       pltpu.VMEM((128,), jnp.int32),
                       pltpu.VMEM((128, d), F32)]
        + [pltpu.SemaphoreType.DMA] * 4
        + [pltpu.VMEM_SHARED((NP, d), F32)],
    )(h2p, dstp, zrows)


def _sc_counts(dstp, ones_h_in, zrows):
    npair = 20

    def body(dst_h, on_h, z_h, out_h, ixa, ixb, onesbuf, saa, sab, acc):
        c = lax.axis_index("c")
        s = lax.axis_index("s")
        base = (s * NC + c) * EPW2
        pltpu.sync_copy(on_h, onesbuf)
        pltpu.sync_copy(z_h, acc.at[pl.ds(s * RPTN, RPTN)])
        plsc.subcore_barrier()

        def step(i, carry):
            offa = base + (2 * i) * 128
            offb = base + (2 * i + 1) * 128

            @pl.when(i > 0)
            def _():
                pltpu.make_async_copy(onesbuf, acc.at[ixa], saa).wait()

            pltpu.sync_copy(dst_h.at[pl.ds(offa, 128)], ixa)
            pltpu.async_copy(onesbuf, acc.at[ixa], saa, add=True)

            @pl.when(i > 0)
            def _():
                pltpu.make_async_copy(onesbuf, acc.at[ixb], sab).wait()

            pltpu.sync_copy(dst_h.at[pl.ds(offb, 128)], ixb)
            pltpu.async_copy(onesbuf, acc.at[ixb], sab, add=True)
            return carry

        lax.fori_loop(0, npair, step, 0)
        pltpu.make_async_copy(onesbuf, acc.at[ixa], saa).wait()
        pltpu.make_async_copy(onesbuf, acc.at[ixb], sab).wait()
        plsc.subcore_barrier()
        pltpu.sync_copy(acc.at[pl.ds(s * RPTN, RPTN)],
                        out_h.at[pl.ds(c * NP + s * RPTN, RPTN)])

    return pl.kernel(
        body,
        out_type=jax.ShapeDtypeStruct((2 * NP, 128), F32),
        mesh=_mesh(),
        scratch_types=[pltpu.VMEM((128,), jnp.int32),
                       pltpu.VMEM((128,), jnp.int32),
                       pltpu.VMEM((128, 128), F32)]
        + [pltpu.SemaphoreType.DMA] * 2
        + [pltpu.VMEM_SHARED((NP, 128), F32)],
    )(dstp, ones_h_in, zrows)


GP = 104


def _sc_pool(h0, h1, h2, batchp, zc, onesp):
    nchunk = NP // 128

    def body(h0_h, h1_h, h2_h, b_h, zc_h, on_h,
             o0_h, o1_h, o2_h, oc_h, idx, b0, b1, b2, ones_v,
             s0, s1, s2, sc, a0, a1, a2, ac):
        c = lax.axis_index("c")
        s = lax.axis_index("s")
        w = s * NC + c

        @pl.when(s == 0)
        def _():
            pltpu.sync_copy(zc_h, a0)
            pltpu.sync_copy(zc_h, a1)
            pltpu.sync_copy(zc_h, a2)
            pltpu.sync_copy(zc_h, ac)

        pltpu.sync_copy(on_h, ones_v)
        plsc.subcore_barrier()
        trip = 2 + (w < nchunk - 2 * NW).astype(jnp.int32)

        def step(j, carry):
            off = (w + NW * j) * 128
            pltpu.sync_copy(b_h.at[pl.ds(off, 128)], idx)
            pltpu.sync_copy(h0_h.at[pl.ds(off, 128)], b0)
            pltpu.sync_copy(h1_h.at[pl.ds(off, 128)], b1)
            pltpu.sync_copy(h2_h.at[pl.ds(off, 128)], b2)
            c0 = pltpu.async_copy(b0, a0.at[idx], s0, add=True)
            c1 = pltpu.async_copy(b1, a1.at[idx], s1, add=True)
            c2 = pltpu.async_copy(b2, a2.at[idx], s2, add=True)
            c3 = pltpu.async_copy(ones_v, ac.at[idx], sc, add=True)
            c0.wait()
            c1.wait()
            c2.wait()
            c3.wait()
            return carry

        lax.fori_loop(0, trip, step, 0)
        plsc.subcore_barrier()

        @pl.when(s == 0)
        def _():
            pltpu.sync_copy(a0, o0_h.at[pl.ds(c * GP, GP)])
            pltpu.sync_copy(a1, o1_h.at[pl.ds(c * GP, GP)])
            pltpu.sync_copy(a2, o2_h.at[pl.ds(c * GP, GP)])
            pltpu.sync_copy(ac, oc_h.at[pl.ds(c * GP, GP)])

    return pl.kernel(
        body,
        out_type=[jax.ShapeDtypeStruct((2 * GP, 128), F32)] * 4,
        mesh=_mesh(),
        scratch_types=[pltpu.VMEM((128,), jnp.int32)]
        + [pltpu.VMEM((128, 128), F32)] * 4
        + [pltpu.SemaphoreType.DMA] * 4
        + [pltpu.VMEM_SHARED((GP, 128), F32)] * 4,
    )(h0, h1, h2, batchp, zc, onesp)


def _stats_update(aff_r, st, i, steps, g_r, b_r, denom):
    @pl.when(i == 0)
    def _():
        aff_r[...] = st

    @pl.when(i > 0)
    def _():
        aff_r[...] = aff_r[...] + st

    @pl.when(i == steps - 1)
    def _():
        a = aff_r[...]
        mu = a[0:1] / denom
        var = a[1:2] / denom - mu * mu
        sc = g_r[...] * lax.rsqrt(var + EPS)
        sh = b_r[...] - mu * sc
        aff_r[...] = jnp.concatenate([sc, sh], 0)


def _tc_addstats(gu, gv, g, b, d):
    be = 4000
    steps = E // be

    def body(gu_r, gv_r, g_r, b_r, y_r, aff_r):
        i = pl.program_id(0)
        y = gu_r[...][:, :d] + gv_r[...][:, :d]
        y_r[...] = y
        st = jnp.concatenate([jnp.sum(y, 0, keepdims=True),
                              jnp.sum(y * y, 0, keepdims=True)], 0)
        _stats_update(aff_r, st, i, steps, g_r, b_r, float(E))

    return pl.pallas_call(
        body,
        grid=(steps,),
        in_specs=[pl.BlockSpec((be, 128), lambda i: (i, 0)),
                  pl.BlockSpec((be, 128), lambda i: (i, 0)),
                  pl.BlockSpec((1, d), lambda i: (0, 0)),
                  pl.BlockSpec((1, d), lambda i: (0, 0))],
        out_specs=[pl.BlockSpec((be, d), lambda i: (i, 0)),
                   pl.BlockSpec((2, d), lambda i: (0, 0))],
        out_shape=[jax.ShapeDtypeStruct((E, d), F32),
                   jax.ShapeDtypeStruct((2, d), F32)],
    )(gu, gv, g.reshape(1, d), b.reshape(1, d))


def _tc_mm2(y1, aff1, w2t, g, b):
    d = y1.shape[1]
    dn = w2t.shape[1]
    be = 4000
    steps = E // be

    def body(y1_r, a1_r, w_r, g_r, b_r, y2_r, aff_r):
        i = pl.program_id(0)
        a = a1_r[...]
        h = jnp.maximum(y1_r[...] * a[0:1] + a[1:2], 0.0)
        y = jnp.dot(h, w_r[...], preferred_element_type=F32)
        y2_r[...] = y
        st = jnp.concatenate([jnp.sum(y, 0, keepdims=True),
                              jnp.sum(y * y, 0, keepdims=True)], 0)
        _stats_update(aff_r, st, i, steps, g_r, b_r, float(E))

    return pl.pallas_call(
        body,
        grid=(steps,),
        in_specs=[pl.BlockSpec((be, d), lambda i: (i, 0)),
                  pl.BlockSpec((2, d), lambda i: (0, 0)),
                  pl.BlockSpec((d, dn), lambda i: (0, 0)),
                  pl.BlockSpec((1, dn), lambda i: (0, 0)),
                  pl.BlockSpec((1, dn), lambda i: (0, 0))],
        out_specs=[pl.BlockSpec((be, dn), lambda i: (i, 0)),
                   pl.BlockSpec((2, dn), lambda i: (0, 0))],
        out_shape=[jax.ShapeDtypeStruct((E, dn), F32),
                   jax.ShapeDtypeStruct((2, dn), F32)],
    )(y1, aff1, w2t, g.reshape(1, dn), b.reshape(1, dn))


def _tc_act(y, aff, pad_to=None):
    m, d = y.shape
    do = pad_to or d
    bm = 4000 if m % 4000 == 0 else 2000
    steps = m // bm

    def body(y_r, a_r, o_r):
        a = a_r[...]
        h = jnp.maximum(y_r[...] * a[0:1] + a[1:2], 0.0)
        if do > d:
            h = jnp.concatenate([h, jnp.zeros((bm, do - d), F32)], axis=1)
        o_r[...] = h

    return pl.pallas_call(
        body,
        grid=(steps,),
        in_specs=[pl.BlockSpec((bm, d), lambda i: (i, 0)),
                  pl.BlockSpec((2, d), lambda i: (0, 0))],
        out_specs=pl.BlockSpec((bm, do), lambda i: (i, 0)),
        out_shape=jax.ShapeDtypeStruct((m, do), F32),
    )(y, aff)


def _agg(p_r, c_r, d):
    pr = p_r[...]
    agg = pr[0, :, :d] + pr[1, :, :d]
    cc = c_r[...]
    cnt = cc[0, :, 0:1] + cc[1, :, 0:1]
    return agg * (1.0 / jnp.maximum(cnt, 1.0))


def _tc_k1(xp, a1t, b1t, ws1t, g1, b1):
    dn = ws1t.shape[1]
    bn = 2000
    steps = N // bn

    def body(x_r, at_r, bt_r, wst_r, g_r, b_r, u_r, v_r, s_r, aff_r):
        i = pl.program_id(0)
        x = x_r[...]
        u_r[...] = jnp.dot(x, at_r[...], preferred_element_type=F32)
        v_r[...] = jnp.dot(x, bt_r[...], preferred_element_type=F32)
        s = jnp.dot(x, wst_r[...], preferred_element_type=F32)
        s_r[...] = s
        st = jnp.concatenate([jnp.sum(s, 0, keepdims=True),
                              jnp.sum(s * s, 0, keepdims=True)], 0)
        _stats_update(aff_r, st, i, steps, g_r, b_r, float(N))

    return pl.pallas_call(
        body,
        grid=(steps,),
        in_specs=[pl.BlockSpec((bn, 8), lambda i: (i, 0)),
                  pl.BlockSpec((8, 128), lambda i: (0, 0)),
                  pl.BlockSpec((8, 128), lambda i: (0, 0)),
                  pl.BlockSpec((8, dn), lambda i: (0, 0)),
                  pl.BlockSpec((1, dn), lambda i: (0, 0)),
                  pl.BlockSpec((1, dn), lambda i: (0, 0))],
        out_specs=[pl.BlockSpec((bn, 128), lambda i: (i, 0)),
                   pl.BlockSpec((bn, 128), lambda i: (i, 0)),
                   pl.BlockSpec((bn, dn), lambda i: (i, 0)),
                   pl.BlockSpec((2, dn), lambda i: (0, 0))],
        out_shape=[jax.ShapeDtypeStruct((N, 128), F32),
                   jax.ShapeDtypeStruct((N, 128), F32),
                   jax.ShapeDtypeStruct((N, dn), F32),
                   jax.ShapeDtypeStruct((2, dn), F32)],
    )(xp, a1t, b1t, ws1t, g1.reshape(1, dn), b1.reshape(1, dn))


def _tc_j_short(p, cnts, s_arr, aff_s, at, bt):
    d = s_arr.shape[1]
    bn = 2000
    steps = N // bn

    def body(p_r, c_r, s_r, a_r, at_r, bt_r, x_r, u_r, v_r):
        a = a_r[...]
        xk = jnp.maximum(s_r[...] * a[0:1] + a[1:2] + _agg(p_r, c_r, d), 0.0)
        x_r[...] = xk
        u_r[...] = jnp.dot(xk, at_r[...], preferred_element_type=F32)
        v_r[...] = jnp.dot(xk, bt_r[...], preferred_element_type=F32)

    return pl.pallas_call(
        body,
        grid=(steps,),
        in_specs=[pl.BlockSpec((2, bn, 128), lambda i: (0, i, 0)),
                  pl.BlockSpec((2, bn, 128), lambda i: (0, i, 0)),
                  pl.BlockSpec((bn, d), lambda i: (i, 0)),
                  pl.BlockSpec((2, d), lambda i: (0, 0)),
                  pl.BlockSpec((d, 128), lambda i: (0, 0)),
                  pl.BlockSpec((d, 128), lambda i: (0, 0))],
        out_specs=[pl.BlockSpec((bn, d), lambda i: (i, 0)),
                   pl.BlockSpec((bn, 128), lambda i: (i, 0)),
                   pl.BlockSpec((bn, 128), lambda i: (i, 0))],
        out_shape=[jax.ShapeDtypeStruct((N, d), F32),
                   jax.ShapeDtypeStruct((N, 128), F32),
                   jax.ShapeDtypeStruct((N, 128), F32)],
    )(p.reshape(2, NP, 128), cnts.reshape(2, NP, 128), s_arr, aff_s, at, bt)


def _tc_j_res(p, cnts, xprev, at, bt, wst, gs, bs):
    d = xprev.shape[1]
    dn = wst.shape[1]
    bn = 2000
    steps = N // bn

    def body(p_r, c_r, xp_r, at_r, bt_r, wst_r, g_r, b_r,
             x_r, u_r, v_r, s_r, aff_r):
        i = pl.program_id(0)
        xk = jnp.maximum(_agg(p_r, c_r, d) + xp_r[...], 0.0)
        x_r[...] = xk
        u_r[...] = jnp.dot(xk, at_r[...], preferred_element_type=F32)
        v_r[...] = jnp.dot(xk, bt_r[...], preferred_element_type=F32)
        s = jnp.dot(xk, wst_r[...], preferred_element_type=F32)
        s_r[...] = s
        st = jnp.concatenate([jnp.sum(s, 0, keepdims=True),
                              jnp.sum(s * s, 0, keepdims=True)], 0)
        _stats_update(aff_r, st, i, steps, g_r, b_r, float(N))

    return pl.pallas_call(
        body,
        grid=(steps,),
        in_specs=[pl.BlockSpec((2, bn, 128), lambda i: (0, i, 0)),
                  pl.BlockSpec((2, bn, 128), lambda i: (0, i, 0)),
                  pl.BlockSpec((bn, d), lambda i: (i, 0)),
                  pl.BlockSpec((d, 128), lambda i: (0, 0)),
                  pl.BlockSpec((d, 128), lambda i: (0, 0)),
                  pl.BlockSpec((d, dn), lambda i: (0, 0)),
                  pl.BlockSpec((1, dn), lambda i: (0, 0)),
                  pl.BlockSpec((1, dn), lambda i: (0, 0))],
        out_specs=[pl.BlockSpec((bn, d), lambda i: (i, 0)),
                   pl.BlockSpec((bn, 128), lambda i: (i, 0)),
                   pl.BlockSpec((bn, 128), lambda i: (i, 0)),
                   pl.BlockSpec((bn, dn), lambda i: (i, 0)),
                   pl.BlockSpec((2, dn), lambda i: (0, 0))],
        out_shape=[jax.ShapeDtypeStruct((N, d), F32),
                   jax.ShapeDtypeStruct((N, 128), F32),
                   jax.ShapeDtypeStruct((N, 128), F32),
                   jax.ShapeDtypeStruct((N, dn), F32),
                   jax.ShapeDtypeStruct((2, dn), F32)],
    )(p.reshape(2, NP, 128), cnts.reshape(2, NP, 128), xprev, at, bt, wst,
      gs.reshape(1, dn), bs.reshape(1, dn))


def _tc_j6(p, cnts, xs, wzs, gz, bz):
    d = xs[4].shape[1]
    dz = wzs[0].shape[1]
    bn = 2000
    steps = N // bn

    def body(p_r, c_r, x1_r, x2_r, x3_r, x4_r, x5_r,
             w1_r, w2_r, w3_r, w4_r, w5_r, w6_r, g_r, b_r, z_r, aff_r):
        i = pl.program_id(0)
        x6 = jnp.maximum(_agg(p_r, c_r, d) + x5_r[...], 0.0)
        z = (jnp.dot(x1_r[...], w1_r[...], preferred_element_type=F32)
             + jnp.dot(x2_r[...], w2_r[...], preferred_element_type=F32)
             + jnp.dot(x3_r[...], w3_r[...], preferred_element_type=F32)
             + jnp.dot(x4_r[...], w4_r[...], preferred_element_type=F32)
             + jnp.dot(x5_r[...], w5_r[...], preferred_element_type=F32)
             + jnp.dot(x6, w6_r[...], preferred_element_type=F32))
        z_r[...] = z
        st = jnp.concatenate([jnp.sum(z, 0, keepdims=True),
                              jnp.sum(z * z, 0, keepdims=True)], 0)
        _stats_update(aff_r, st, i, steps, g_r, b_r, float(N))

    x1, x2, x3, x4, x5 = xs
    ds = [x.shape[1] for x in xs]
    return pl.pallas_call(
        body,
        grid=(steps,),
        in_specs=[pl.BlockSpec((2, bn, 128), lambda i: (0, i, 0)),
                  pl.BlockSpec((2, bn, 128), lambda i: (0, i, 0))]
        + [pl.BlockSpec((bn, dd), lambda i: (i, 0)) for dd in ds]
        + [pl.BlockSpec((dd, dz), lambda i: (0, 0)) for dd in ds + [d]]
        + [pl.BlockSpec((1, dz), lambda i: (0, 0)),
           pl.BlockSpec((1, dz), lambda i: (0, 0))],
        out_specs=[pl.BlockSpec((bn, dz), lambda i: (i, 0)),
                   pl.BlockSpec((2, dz), lambda i: (0, 0))],
        out_shape=[jax.ShapeDtypeStruct((N, dz), F32),
                   jax.ShapeDtypeStruct((2, dz), F32)],
    )(p.reshape(2, NP, 128), cnts.reshape(2, NP, 128), x1, x2, x3, x4, x5,
      *wzs, gz.reshape(1, dz), bz.reshape(1, dz))


def _tc_head(pps, pc, w2t, b2, wlt, bl):
    def body(p0_r, p1_r, p2_r, pc_r, w2_r, b2_r, wl_r, bl_r, out_r):
        pr = jnp.concatenate([p0_r[...], p1_r[...], p2_r[...]], axis=2)
        cc = pc_r[...]
        cnt = cc[0, :, 0:1] + cc[1, :, 0:1]
        g = (pr[0] + pr[1]) * (1.0 / jnp.maximum(cnt, 1.0))
        h2 = jnp.maximum(jnp.dot(g, w2_r[...], preferred_element_type=F32)
                         + b2_r[...], 0.0)
        o = jnp.dot(h2, wl_r[...], preferred_element_type=F32) + bl_r[...]
        m = jnp.max(o[:, :1], axis=1, keepdims=True)
        e = jnp.exp(o - m)
        out_r[...] = e / jnp.sum(e[:, :1], axis=1, keepdims=True)

    return pl.pallas_call(
        body,
        out_shape=jax.ShapeDtypeStruct((GP, 128), F32),
    )(pps[0].reshape(2, GP, 128), pps[1].reshape(2, GP, 128),
      pps[2].reshape(2, GP, 128), pc.reshape(2, GP, 128), w2t, b2, wlt, bl)


def _edge_layer(u, v, src, dst, dstp, p, zrows):
    d = p["lin2"]["w"].shape[0]
    gu, gv = _sc_gather(u, v, src, dst)
    y1, aff1 = _tc_addstats(gu, gv, p["bn1"]["g"], p["bn1"]["b"], d)
    y2, aff2 = _tc_mm2(y1, aff1, p["lin2"]["w"].T, p["bn2"]["g"], p["bn2"]["b"])
    h2 = _tc_act(y2, aff2, pad_to=128)
    h2p = jnp.pad(h2, ((0, EP - E), (0, 0)))
    return _sc_scatter(h2p, dstp, zrows)


def _split_w1(p, din):
    w1 = p["lin1"]["w"]
    dout = w1.shape[0]
    a = w1[:, :din]
    b = w1[:, din:]
    pad = ((0, 0), (0, 128 - dout))
    return jnp.pad((a - b).T, pad), jnp.pad(b.T, pad)


def kernel(x, params, edge_index, batch):
    src = edge_index[0]
    dst = edge_index[1]

    z128 = jnp.zeros((RPTN, 128), F32)
    dstp = jnp.pad(dst, (0, EP - E), constant_values=N)

    cnts = _sc_counts(dstp, jnp.ones((128, 128), F32), z128)

    xp = jnp.pad(x, ((0, 0), (0, 5)))
    a1t, b1t = _split_w1(params["conv1"], 3)
    a1t = jnp.pad(a1t, ((0, 5), (0, 0)))
    b1t = jnp.pad(b1t, ((0, 5), (0, 0)))
    ws1t = jnp.pad(params["sc1"]["lin"]["w"].T, ((0, 5), (0, 0)))
    u1, v1, s1, aff_s1 = _tc_k1(xp, a1t, b1t, ws1t,
                                params["sc1"]["bn"]["g"],
                                params["sc1"]["bn"]["b"])
    p1 = _edge_layer(u1, v1, src, dst, dstp, params["conv1"], z128)

    a2t, b2t = _split_w1(params["conv2"], 32)
    x1, u2, v2 = _tc_j_short(p1, cnts, s1, aff_s1, a2t, b2t)
    p2 = _edge_layer(u2, v2, src, dst, dstp, params["conv2"], z128)

    a3t, b3t = _split_w1(params["conv3"], 32)
    x2, u3, v3, s3, aff_s3 = _tc_j_res(p2, cnts, x1, a3t, b3t,
                                       params["sc3"]["lin"]["w"].T,
                                       params["sc3"]["bn"]["g"],
                                       params["sc3"]["bn"]["b"])
    p3 = _edge_layer(u3, v3, src, dst, dstp, params["conv3"], z128)

    a4t, b4t = _split_w1(params["conv4"], 64)
    x3, u4, v4 = _tc_j_short(p3, cnts, s3, aff_s3, a4t, b4t)
    p4 = _edge_layer(u4, v4, src, dst, dstp, params["conv4"], z128)

    a5t, b5t = _split_w1(params["conv5"], 64)
    x4, u5, v5, s5, aff_s5 = _tc_j_res(p4, cnts, x3, a5t, b5t,
                                       params["sc5"]["lin"]["w"].T,
                                       params["sc5"]["bn"]["g"],
                                       params["sc5"]["bn"]["b"])
    p5 = _edge_layer(u5, v5, src, dst, dstp, params["conv5"], z128)

    a6t, b6t = _split_w1(params["conv6"], 128)
    x5, u6, v6 = _tc_j_short(p5, cnts, s5, aff_s5, a6t, b6t)
    p6 = _edge_layer(u6, v6, src, dst, dstp, params["conv6"], z128)

    wseq = params["seq1"]["lin"]["w"]
    offs = [0, 32, 64, 128, 192, 320, 448]
    wzs = [wseq[:, offs[k]:offs[k + 1]].T for k in range(6)]
    z, aff_z = _tc_j6(p6, cnts, [x1, x2, x3, x4, x5], wzs,
                      params["seq1"]["bn"]["g"], params["seq1"]["bn"]["b"])
    h = _tc_act(z, aff_z)

    hp = jnp.pad(h, ((0, NP - N), (0, 0)))
    batchp = jnp.pad(batch, (0, NP - N), constant_values=G)
    zc = jnp.zeros((GP, 128), F32)
    onesp = jnp.ones((128, 128), F32)
    pp0, pp1, pp2, pc = _sc_pool(hp[:, :128], hp[:, 128:256], hp[:, 256:],
                                 batchp, zc, onesp)

    w2t = params["seq2"]["lin"]["w"].T
    b2 = params["seq2"]["lin"]["b"].reshape(1, 256)
    wlt = jnp.pad(params["lin"]["w"].T, ((0, 0), (0, 127)))
    bl = jnp.pad(params["lin"]["b"].reshape(1, 1), ((0, 0), (0, 127)))
    out = _tc_head((pp0, pp1, pp2), pc, w2t, b2, wlt, bl)
    return out[:G, :1]
